# baseline (speedup 1.0000x reference)
import functools

import jax
import jax.numpy as jnp
from jax import lax
from jax.experimental import pallas as pl
from jax.experimental.pallas import tpu as pltpu
from jax.experimental.pallas import tpu_sc as plsc

N_CORES = 2
N_SUBCORES = 16
N_WORKERS = N_CORES * N_SUBCORES

D = 1024
B = 4 * 8192
B_PER_W = B // N_WORKERS
NSEM = 16


def _sc_gather(table, idx):
    mesh = plsc.VectorSubcoreMesh(core_axis_name="c", subcore_axis_name="s")

    @functools.partial(
        pl.kernel,
        mesh=mesh,
        out_type=jax.ShapeDtypeStruct((B, D), jnp.float32),
        scratch_types=[
            pltpu.VMEM((B_PER_W,), jnp.int32),
        ]
        + [pltpu.SemaphoreType.DMA] * NSEM,
    )
    def k(table_hbm, idx_hbm, out_hbm, idx_s, *sems):
        wid = lax.axis_index("s") * N_CORES + lax.axis_index("c")
        base = wid * B_PER_W
        pltpu.sync_copy(idx_hbm.at[pl.ds(base, B_PER_W)], idx_s)

        def rcopy(row, j, s):
            return pltpu.make_async_copy(
                table_hbm.at[pl.ds(row, 1)],
                out_hbm.at[pl.ds(base + j, 1)],
                sems[s],
            )

        def block(blk, first):
            vec = idx_s[pl.ds(blk * NSEM, NSEM)]
            for kk in range(NSEM):
                if not first:
                    rcopy(0, 0, kk).wait()
                rcopy(vec[kk], blk * NSEM + kk, kk).start()

        block(0, True)

        @pl.loop(1, B_PER_W // NSEM)
        def _(blk):
            block(blk, False)

        for m in range(NSEM):
            rcopy(0, 0, m).wait()

    return k(table, idx)


def kernel(positions, positional_embeddings):
    idx = positions.reshape(-1).astype(jnp.int32)
    out = _sc_gather(positional_embeddings, idx)
    return out.reshape(positions.shape + (positional_embeddings.shape[1],))


# speedup vs baseline: 35.3947x; 35.3947x over previous
import functools

import jax
import jax.numpy as jnp
from jax import lax
from jax.experimental import pallas as pl
from jax.experimental.pallas import tpu as pltpu
from jax.experimental.pallas import tpu_sc as plsc

N_CORES = 2
N_SUBCORES = 16
N_WORKERS = N_CORES * N_SUBCORES

D = 1024
B = 4 * 8192
B_PER_W = B // N_WORKERS
CHUNK = 32
NBUF = 2
N_CHUNKS = B_PER_W // CHUNK


def _sc_gather(table, idx):
    mesh = plsc.VectorSubcoreMesh(core_axis_name="c", subcore_axis_name="s")

    @functools.partial(
        pl.kernel,
        mesh=mesh,
        out_type=jax.ShapeDtypeStruct((B, D), jnp.float32),
        scratch_types=[
            pltpu.VMEM((B_PER_W,), jnp.int32),
            pltpu.VMEM((NBUF, CHUNK, D), jnp.float32),
        ]
        + [pltpu.SemaphoreType.DMA] * (2 * NBUF),
    )
    def k(table_hbm, idx_hbm, out_hbm, idx_v, rows_v, *sems):
        gsem, wsem = sems[:NBUF], sems[NBUF:]
        wid = lax.axis_index("s") * N_CORES + lax.axis_index("c")
        base = wid * B_PER_W
        pltpu.sync_copy(idx_hbm.at[pl.ds(base, B_PER_W)], idx_v)

        def gcopy(i, s):
            return pltpu.make_async_copy(
                table_hbm.at[idx_v.at[pl.ds(i * CHUNK, CHUNK)]],
                rows_v.at[s],
                gsem[s],
            )

        def wcopy(i, s):
            return pltpu.make_async_copy(
                rows_v.at[s],
                out_hbm.at[pl.ds(base + i * CHUNK, CHUNK)],
                wsem[s],
            )

        for m in range(NBUF - 1):
            gcopy(m, m).start()
        gcopy(0, 0).wait()
        wcopy(0, 0).start()
        gcopy(NBUF - 1, NBUF - 1).start()

        @pl.loop(0, (N_CHUNKS - NBUF) // NBUF)
        def _(blk):
            ibase = 1 + blk * NBUF
            for kk in range(NBUF):
                i = ibase + kk
                s = (1 + kk) % NBUF
                sp = (s - 1) % NBUF
                gcopy(i, s).wait()
                wcopy(i, s).start()
                wcopy(i - 1, sp).wait()
                gcopy(i + NBUF - 1, sp).start()

        for i in range(N_CHUNKS - NBUF + 1, N_CHUNKS):
            s = i % NBUF
            gcopy(i, s).wait()
            wcopy(i, s).start()
        for i in range(N_CHUNKS - NBUF, N_CHUNKS):
            wcopy(i, i % NBUF).wait()

    return k(table, idx)


def kernel(positions, positional_embeddings):
    idx = positions.reshape(-1).astype(jnp.int32)
    out = _sc_gather(positional_embeddings, idx)
    return out.reshape(positions.shape + (positional_embeddings.shape[1],))
